# baseline (speedup 1.0000x reference)
import functools

import jax
import jax.numpy as jnp
from jax import lax
from jax.experimental import pallas as pl
from jax.experimental.pallas import tpu as pltpu
from jax.experimental.pallas import tpu_sc as plsc

N = 1024
E = 32768
EDGE_DIM = 16
L = 5
NN = N * N
NW = 32
P = NN // NW
ROWS = N // NW
LANES = 16

CP = 4096
NCHUNK = P // CP
CHUNK_W = CP * L
STAGE_OFF = CHUNK_W

F32_MAX = 3.4028235e38


def _tc_table_body(w_ref, a_ref, o_ref):
    t = lax.dot_general(
        w_ref[...], a_ref[...], (((1,), (1,)), ((), ())),
        preferred_element_type=jnp.float32)
    o_ref[...] = lax.bitcast_convert_type(t, jnp.int32)


def _build_table(w_pad, edge_attr):
    return pl.pallas_call(
        _tc_table_body,
        out_shape=jax.ShapeDtypeStruct((8, E), jnp.int32),
    )(w_pad, edge_attr)


def _sc_body(tbl_hbm, ep_hbm, out_hbm, idxt_hbm, buf_v, acc_v):
    wid = lax.axis_index("s") * 2 + lax.axis_index("c")
    base = wid * P
    row0 = wid * ROWS
    iota = lax.iota(jnp.int32, LANES)

    for c in range(NCHUNK):
        pltpu.sync_copy(
            ep_hbm.at[pl.ds((base + c * CP) * L, CHUNK_W)],
            buf_v.at[pl.ds(0, CHUNK_W)])
        for l in range(L):
            start = iota * L + l

            def ext_body(v, ivec, l=l):
                ex = plsc.load_gather(buf_v, [ivec])
                buf_v[pl.ds(STAGE_OFF + l * CP + v * LANES, LANES)] = ex
                return ivec + LANES * L

            lax.fori_loop(0, CP // LANES, ext_body, start)
        for l in range(L):
            pltpu.sync_copy(
                buf_v.at[pl.ds(STAGE_OFF + l * CP, CP)],
                idxt_hbm.at[pl.ds(l * NN + base + c * CP, CP)])

    for l in range(L):
        pltpu.sync_copy(tbl_hbm.at[pl.ds(l * E, E)], buf_v.at[pl.ds(0, E)])
        pltpu.sync_copy(idxt_hbm.at[pl.ds(l * NN + base, P)],
                        buf_v.at[pl.ds(E, P)])

        def body(i, _, l=l):
            r = i >> 6
            k = (i & 63) * LANES
            iv = buf_v[pl.ds(E + i * LANES, LANES)]
            g = plsc.bitcast(plsc.load_gather(buf_v, [iv]), jnp.float32)
            if l == 0:
                acc_v[r, pl.ds(k, LANES)] = g
            elif l == L - 1:
                s = acc_v[r, pl.ds(k, LANES)] + g
                s = jnp.clip(s, -F32_MAX, F32_MAX)
                acc_v[r, pl.ds(k, LANES)] = jnp.where(s != s, 0.0, s)
            else:
                acc_v[r, pl.ds(k, LANES)] = acc_v[r, pl.ds(k, LANES)] + g
            return _

        lax.fori_loop(0, P // LANES, body, 0)

    pltpu.sync_copy(acc_v, out_hbm.at[pl.ds(row0, ROWS), :])


@functools.partial(
    pl.kernel,
    mesh=plsc.VectorSubcoreMesh(core_axis_name="c", subcore_axis_name="s"),
    out_type=(
        jax.ShapeDtypeStruct((N, N), jnp.float32),
        jax.ShapeDtypeStruct((L * NN,), jnp.int32),
    ),
    compiler_params=pltpu.CompilerParams(needs_layout_passes=False),
    scratch_types=[
        pltpu.VMEM((E + P,), jnp.int32),
        pltpu.VMEM((ROWS, N), jnp.float32),
    ],
)
def _sc_gather(tbl_hbm, ep_hbm, out_hbm, idxt_hbm, buf_v, acc_v):
    _sc_body(tbl_hbm, ep_hbm, out_hbm, idxt_hbm, buf_v, acc_v)


def kernel(x, edge_attr, edge_paths, edge_weights):
    del x
    w_pad = jnp.zeros((8, EDGE_DIM), jnp.float32).at[:L].set(
        edge_weights.astype(jnp.float32) / L)
    tbl = _build_table(w_pad, edge_attr).reshape(8 * E)
    ep_flat = edge_paths.astype(jnp.int32).reshape(NN * L)
    out, _ = _sc_gather(tbl, ep_flat)
    return out


# speedup vs baseline: 5.2097x; 5.2097x over previous
import functools

import jax
import jax.numpy as jnp
from jax import lax
from jax.experimental import pallas as pl
from jax.experimental.pallas import tpu as pltpu
from jax.experimental.pallas import tpu_sc as plsc

N = 1024
E = 32768
EDGE_DIM = 16
L = 5
NN = N * N
NW = 32
P = NN // NW
ROWS = N // NW
LANES = 16

F32_MAX = 3.4028235e38


def _tc_table_body(w_ref, a_ref, *o_refs):
    t = lax.dot_general(
        w_ref[...], a_ref[...], (((1,), (1,)), ((), ())),
        preferred_element_type=jnp.float32)
    for l in range(L):
        o_refs[l][...] = t[l]


def _build_table(w_pad, edge_attr):
    return pl.pallas_call(
        _tc_table_body,
        out_shape=[jax.ShapeDtypeStruct((E,), jnp.float32)] * L,
    )(w_pad, edge_attr)


def _sc_body(t0, t1, t2, t3, t4, ept_hbm, out_hbm, tbl_v, idx_v, acc_v):
    wid = lax.axis_index("s") * 2 + lax.axis_index("c")
    row0 = wid * ROWS

    for l, t_hbm in enumerate((t0, t1, t2, t3, t4)):
        pltpu.sync_copy(t_hbm, tbl_v)
        pltpu.sync_copy(ept_hbm.at[pl.ds(l, 1), pl.ds(row0, ROWS), :], idx_v)

        def body(i, _, l=l):
            r = i >> 6
            k = (i & 63) * LANES
            iv = idx_v[0, r, pl.ds(k, LANES)]
            g = plsc.load_gather(tbl_v, [iv])
            if l == 0:
                acc_v[r, pl.ds(k, LANES)] = g
            elif l == L - 1:
                s = acc_v[r, pl.ds(k, LANES)] + g
                s = jnp.clip(s, -F32_MAX, F32_MAX)
                acc_v[r, pl.ds(k, LANES)] = jnp.where(s != s, 0.0, s)
            else:
                acc_v[r, pl.ds(k, LANES)] = acc_v[r, pl.ds(k, LANES)] + g
            return _

        lax.fori_loop(0, P // LANES, body, 0)

    pltpu.sync_copy(acc_v, out_hbm.at[pl.ds(row0, ROWS), :])


@functools.partial(
    pl.kernel,
    mesh=plsc.VectorSubcoreMesh(core_axis_name="c", subcore_axis_name="s"),
    out_type=jax.ShapeDtypeStruct((N, N), jnp.float32),
    compiler_params=pltpu.CompilerParams(needs_layout_passes=False),
    scratch_types=[
        pltpu.VMEM((E,), jnp.float32),
        pltpu.VMEM((1, ROWS, N), jnp.int32),
        pltpu.VMEM((ROWS, N), jnp.float32),
    ],
)
def _sc_gather(t0, t1, t2, t3, t4, ept_hbm, out_hbm, tbl_v, idx_v, acc_v):
    _sc_body(t0, t1, t2, t3, t4, ept_hbm, out_hbm, tbl_v, idx_v, acc_v)


def kernel(x, edge_attr, edge_paths, edge_weights):
    del x
    w_pad = jnp.zeros((8, EDGE_DIM), jnp.float32).at[:L].set(
        edge_weights.astype(jnp.float32) / L)
    tables = _build_table(w_pad, edge_attr)
    ept = jnp.transpose(edge_paths.astype(jnp.int32), (2, 0, 1))
    return _sc_gather(*tables, ept)
